# baseline (device time: 66442 ns/iter reference)
import jax
import jax.numpy as jnp
from jax import lax
from jax.experimental import pallas as pl
from jax.experimental.pallas import tpu as pltpu

N_DEV = 8
B = 2
SQ = 128
D = 512
HQ_LOCAL = 8
DH = 64
GROUP = 4
KV_LOCAL = HQ_LOCAL // GROUP


def _body(x_ref, wq_ref, wo_ref, wk_ref, wv_ref, out_ref,
          comm_ref, send_sems, recv_sems):
    my = lax.axis_index("i")
    left = lax.rem(my + N_DEV - 1, N_DEV)
    right = lax.rem(my + 1, N_DEV)

    barrier_sem = pltpu.get_barrier_semaphore()
    for nbr in (left, right):
        pl.semaphore_signal(barrier_sem, inc=1, device_id=(nbr,),
                            device_id_type=pl.DeviceIdType.MESH)
    pl.semaphore_wait(barrier_sem, 2)

    wq = wq_ref[...]
    wo = wo_ref[...]
    wk = wk_ref[...]
    wv = wv_ref[...]
    for b in range(B):
        xb = x_ref[b]
        q = jnp.dot(xb, wq, preferred_element_type=jnp.float32)
        k = jnp.dot(xb, wk, preferred_element_type=jnp.float32)
        v = jnp.dot(xb, wv, preferred_element_type=jnp.float32)
        heads = []
        for g in range(HQ_LOCAL):
            j = g // GROUP
            qh = q[:, g * DH:(g + 1) * DH]
            kh = k[:, j * DH:(j + 1) * DH]
            vh = v[:, j * DH:(j + 1) * DH]
            s = lax.dot_general(qh, kh, (((1,), (1,)), ((), ())),
                                preferred_element_type=jnp.float32) * 0.125
            m = jnp.max(s, axis=1, keepdims=True)
            p = jnp.exp(s - m)
            l = jnp.sum(p, axis=1, keepdims=True)
            heads.append(jnp.dot(p, vh, preferred_element_type=jnp.float32) / l)
        attn = jnp.concatenate(heads, axis=1)
        partial = jnp.dot(attn, wo, preferred_element_type=jnp.float32)
        out_ref[b] = partial
        comm_ref[0, b] = partial

    for h in range(N_DEV - 1):
        rdma = pltpu.make_async_remote_copy(
            src_ref=comm_ref.at[h],
            dst_ref=comm_ref.at[h + 1],
            send_sem=send_sems.at[h],
            recv_sem=recv_sems.at[h],
            device_id=(right,),
            device_id_type=pl.DeviceIdType.MESH,
        )
        rdma.start()
        rdma.wait()
        out_ref[...] = out_ref[...] + comm_ref[h + 1]


def kernel(x, Wq, Wo, Wk, Wv):
    my = lax.axis_index("i")
    kv_cols = KV_LOCAL * DH
    wk_s = lax.dynamic_slice(Wk, (0, my * kv_cols), (D, kv_cols))
    wv_s = lax.dynamic_slice(Wv, (0, my * kv_cols), (D, kv_cols))

    if hasattr(pltpu, "CompilerParams"):
        params = pltpu.CompilerParams(collective_id=0)
    else:
        params = pltpu.TPUCompilerParams(collective_id=0)

    return pl.pallas_call(
        _body,
        out_shape=jax.ShapeDtypeStruct((B, SQ, D), jnp.float32),
        in_specs=[pl.BlockSpec(memory_space=pltpu.VMEM)] * 5,
        out_specs=pl.BlockSpec(memory_space=pltpu.VMEM),
        scratch_shapes=[
            pltpu.VMEM((N_DEV, B, SQ, D), jnp.float32),
            pltpu.SemaphoreType.DMA((N_DEV - 1,)),
            pltpu.SemaphoreType.DMA((N_DEV - 1,)),
        ],
        compiler_params=params,
    )(x, Wq, Wo, wk_s, wv_s)


# device time: 37113 ns/iter; 1.7903x vs baseline; 1.7903x over previous
import jax
import jax.numpy as jnp
from jax import lax
from jax.experimental import pallas as pl
from jax.experimental.pallas import tpu as pltpu

N_DEV = 8
B = 2
SQ = 128
D = 512
HQ_LOCAL = 8
DH = 64
GROUP = 4
KV_LOCAL = HQ_LOCAL // GROUP


def _body(x_ref, wq_ref, wo_ref, wk_ref, wv_ref, out_ref,
          comm_ref, send_sems, recv_sems):
    my = lax.axis_index("i")
    partners = [jnp.bitwise_xor(my, 1 << k) for k in range(3)]

    barrier_sem = pltpu.get_barrier_semaphore()
    for nbr in partners:
        pl.semaphore_signal(barrier_sem, inc=1, device_id=(nbr,),
                            device_id_type=pl.DeviceIdType.MESH)
    pl.semaphore_wait(barrier_sem, 3)

    wq = wq_ref[...]
    wo = wo_ref[...]
    wk = wk_ref[...]
    wv = wv_ref[...]
    for b in range(B):
        xb = x_ref[b]
        q = jnp.dot(xb, wq, preferred_element_type=jnp.float32)
        k = jnp.dot(xb, wk, preferred_element_type=jnp.float32)
        v = jnp.dot(xb, wv, preferred_element_type=jnp.float32)
        heads = []
        for g in range(HQ_LOCAL):
            j = g // GROUP
            qh = q[:, g * DH:(g + 1) * DH]
            kh = k[:, j * DH:(j + 1) * DH]
            vh = v[:, j * DH:(j + 1) * DH]
            s = lax.dot_general(qh, kh, (((1,), (1,)), ((), ())),
                                preferred_element_type=jnp.float32) * 0.125
            m = jnp.max(s, axis=1, keepdims=True)
            p = jnp.exp(s - m)
            l = jnp.sum(p, axis=1, keepdims=True)
            heads.append(jnp.dot(p, vh, preferred_element_type=jnp.float32) / l)
        attn = jnp.concatenate(heads, axis=1)
        partial = jnp.dot(attn, wo, preferred_element_type=jnp.float32)
        out_ref[b] = partial

    for k in range(3):
        rdma = pltpu.make_async_remote_copy(
            src_ref=out_ref,
            dst_ref=comm_ref.at[k],
            send_sem=send_sems.at[k],
            recv_sem=recv_sems.at[k],
            device_id=(partners[k],),
            device_id_type=pl.DeviceIdType.MESH,
        )
        rdma.start()
        rdma.wait()
        out_ref[...] = out_ref[...] + comm_ref[k]


def kernel(x, Wq, Wo, Wk, Wv):
    my = lax.axis_index("i")
    kv_cols = KV_LOCAL * DH
    wk_s = lax.dynamic_slice(Wk, (0, my * kv_cols), (D, kv_cols))
    wv_s = lax.dynamic_slice(Wv, (0, my * kv_cols), (D, kv_cols))

    if hasattr(pltpu, "CompilerParams"):
        params = pltpu.CompilerParams(collective_id=0)
    else:
        params = pltpu.TPUCompilerParams(collective_id=0)

    return pl.pallas_call(
        _body,
        out_shape=jax.ShapeDtypeStruct((B, SQ, D), jnp.float32),
        in_specs=[pl.BlockSpec(memory_space=pltpu.VMEM)] * 5,
        out_specs=pl.BlockSpec(memory_space=pltpu.VMEM),
        scratch_shapes=[
            pltpu.VMEM((3, B, SQ, D), jnp.float32),
            pltpu.SemaphoreType.DMA((3,)),
            pltpu.SemaphoreType.DMA((3,)),
        ],
        compiler_params=params,
    )(x, Wq, Wo, wk_s, wv_s)


# device time: 26056 ns/iter; 2.5500x vs baseline; 1.4244x over previous
import jax
import jax.numpy as jnp
from jax import lax
from jax.experimental import pallas as pl
from jax.experimental.pallas import tpu as pltpu

N_DEV = 8
B = 2
SQ = 128
D = 512
HALF = D // 2
HQ_LOCAL = 8
DH = 64
GROUP = 4
KV_LOCAL = HQ_LOCAL // GROUP


def _body(x_ref, wq_ref, wo_ref, wk_ref, wv_ref, out_ref,
          comm_ref, send_ref, send_sems, recv_sems):
    my = lax.axis_index("i")
    partners = [jnp.bitwise_xor(my, 1 << k) for k in range(3)]

    barrier_sem = pltpu.get_barrier_semaphore()
    for nbr in partners:
        pl.semaphore_signal(barrier_sem, inc=1, device_id=(nbr,),
                            device_id_type=pl.DeviceIdType.MESH)
    pl.semaphore_wait(barrier_sem, 3)

    wqkv = jnp.concatenate([wq_ref[...], wk_ref[...], wv_ref[...]], axis=1)
    attn = []
    for b in range(B):
        qkv = jnp.dot(x_ref[b], wqkv, preferred_element_type=jnp.float32)
        kv0 = D
        heads = []
        for g in range(HQ_LOCAL):
            j = g // GROUP
            qh = qkv[:, g * DH:(g + 1) * DH]
            kh = qkv[:, kv0 + j * DH:kv0 + (j + 1) * DH]
            vh = qkv[:, kv0 + (KV_LOCAL + j) * DH:kv0 + (KV_LOCAL + j + 1) * DH]
            s = lax.dot_general(qh, kh, (((1,), (1,)), ((), ())),
                                preferred_element_type=jnp.float32) * 0.125
            m = jnp.max(s, axis=1, keepdims=True)
            p = jnp.exp(s - m)
            l = jnp.sum(p, axis=1, keepdims=True)
            heads.append(jnp.dot(p, vh, preferred_element_type=jnp.float32) / l)
        attn.append(jnp.concatenate(heads, axis=1))

    def make_rdma(k, h):
        r = 2 * k + h
        return pltpu.make_async_remote_copy(
            src_ref=send_ref.at[r],
            dst_ref=comm_ref.at[r],
            send_sem=send_sems.at[r],
            recv_sem=recv_sems.at[r],
            device_id=(partners[k],),
            device_id_type=pl.DeviceIdType.MESH,
        )

    rdmas = {}
    wo = wo_ref[...]
    for h in range(2):
        for b in range(B):
            ph = jnp.dot(attn[b], wo[:, h * HALF:(h + 1) * HALF],
                         preferred_element_type=jnp.float32)
            out_ref[b, :, h * HALF:(h + 1) * HALF] = ph
            send_ref[h, b] = ph.astype(jnp.bfloat16)
        rdmas[(0, h)] = make_rdma(0, h)
        rdmas[(0, h)].start()

    for k in range(3):
        for h in range(2):
            rdmas[(k, h)].wait_recv()
            acc = (out_ref[:, :, h * HALF:(h + 1) * HALF]
                   + comm_ref[2 * k + h].astype(jnp.float32))
            out_ref[:, :, h * HALF:(h + 1) * HALF] = acc
            if k < 2:
                send_ref[2 * (k + 1) + h] = acc.astype(jnp.bfloat16)
                rdmas[(k + 1, h)] = make_rdma(k + 1, h)
                rdmas[(k + 1, h)].start()

    for k in range(3):
        for h in range(2):
            rdmas[(k, h)].wait_send()


def kernel(x, Wq, Wo, Wk, Wv):
    my = lax.axis_index("i")
    kv_cols = KV_LOCAL * DH
    wk_s = lax.dynamic_slice(Wk, (0, my * kv_cols), (D, kv_cols))
    wv_s = lax.dynamic_slice(Wv, (0, my * kv_cols), (D, kv_cols))

    if hasattr(pltpu, "CompilerParams"):
        params = pltpu.CompilerParams(collective_id=0)
    else:
        params = pltpu.TPUCompilerParams(collective_id=0)

    return pl.pallas_call(
        _body,
        out_shape=jax.ShapeDtypeStruct((B, SQ, D), jnp.float32),
        in_specs=[pl.BlockSpec(memory_space=pltpu.VMEM)] * 5,
        out_specs=pl.BlockSpec(memory_space=pltpu.VMEM),
        scratch_shapes=[
            pltpu.VMEM((6, B, SQ, HALF), jnp.bfloat16),
            pltpu.VMEM((6, B, SQ, HALF), jnp.bfloat16),
            pltpu.SemaphoreType.DMA((6,)),
            pltpu.SemaphoreType.DMA((6,)),
        ],
        compiler_params=params,
    )(x, Wq, Wo, wk_s, wv_s)
